# baseline (speedup 1.0000x reference)
import functools

import jax
import jax.numpy as jnp
from jax import lax
from jax.experimental import pallas as pl
from jax.experimental.pallas import tpu as pltpu
from jax.experimental.pallas import tpu_sc as plsc


def _make_sc_gather(n_rows: int, emb: int, num_cores: int, num_subcores: int):
    nw = num_cores * num_subcores
    rows_per_w = max(8, n_rows // nw)
    n_active = n_rows // rows_per_w
    mesh = plsc.VectorSubcoreMesh(core_axis_name="c", subcore_axis_name="s")

    @functools.partial(
        pl.kernel,
        mesh=mesh,
        out_type=jax.ShapeDtypeStruct((n_rows, emb), jnp.float32),
        scratch_types=[
            pltpu.VMEM((rows_per_w,), jnp.int32),
            pltpu.VMEM((rows_per_w, emb), jnp.float32),
            pltpu.SemaphoreType.DMA,
        ],
        compiler_params=pltpu.CompilerParams(use_tc_tiling_on_sc=False),
    )
    def gather(idx_hbm, table_hbm, pe_hbm, idx_v, rows_v, sem):
        wid = lax.axis_index("s") * num_cores + lax.axis_index("c")

        @pl.when(wid < n_active)
        def _():
            base = wid * rows_per_w
            pltpu.sync_copy(idx_hbm.at[pl.ds(base, rows_per_w)], idx_v)
            pltpu.async_copy(table_hbm.at[idx_v], rows_v, sem).wait()
            pltpu.sync_copy(rows_v, pe_hbm.at[pl.ds(base, rows_per_w)])

    return gather


def _make_add(rows: int, e: int, n: int, ch: int, nbuf: int):
    nchunks = rows // ch

    def body(x_hbm, pe_vmem, o_hbm, inb, outb, insems, outsems):
        def in_copy(c):
            slot = c % nbuf
            return pltpu.make_async_copy(
                x_hbm.at[pl.ds(c * ch, ch)], inb.at[slot], insems.at[slot])

        def out_copy(c):
            slot = c % nbuf
            return pltpu.make_async_copy(
                outb.at[slot], o_hbm.at[pl.ds(c * ch, ch)], outsems.at[slot])

        for c in range(nbuf):
            in_copy(c).start()
        for c in range(nchunks):
            slot = c % nbuf
            in_copy(c).wait()
            if c >= nbuf:
                out_copy(c - nbuf).wait()
            outb[slot] = inb[slot] + pe_vmem[...][None, :, :]
            out_copy(c).start(priority=1)
            if c + nbuf < nchunks:
                in_copy(c + nbuf).start()
        for c in range(nchunks - nbuf, nchunks):
            out_copy(c).wait()

    return body


def kernel(x, input_channels, pos_embed):
    B, N, P, E = x.shape
    input_channels = input_channels.astype(jnp.int32)

    info = plsc.get_sparse_core_info()
    gather = _make_sc_gather(N, E, info.num_cores, info.num_subcores)
    pe = gather(input_channels, pos_embed)

    xt = jnp.transpose(x, (0, 2, 3, 1)).reshape(B * P, E, N)
    pe_t = pe.T

    CH = 16
    NBUF = 12
    out_t = pl.pallas_call(
        _make_add(B * P, E, N, CH, NBUF),
        in_specs=[
            pl.BlockSpec(memory_space=pltpu.MemorySpace.HBM),
            pl.BlockSpec(memory_space=pltpu.MemorySpace.VMEM),
        ],
        out_specs=pl.BlockSpec(memory_space=pltpu.MemorySpace.HBM),
        out_shape=jax.ShapeDtypeStruct((B * P, E, N), jnp.float32),
        scratch_shapes=[
            pltpu.VMEM((NBUF, CH, E, N), jnp.float32),
            pltpu.VMEM((NBUF, CH, E, N), jnp.float32),
            pltpu.SemaphoreType.DMA((NBUF,)),
            pltpu.SemaphoreType.DMA((NBUF,)),
        ],
    )(xt, pe_t)
    return jnp.transpose(out_t.reshape(B, P, E, N), (0, 3, 1, 2))


# speedup vs baseline: 1.0014x; 1.0014x over previous
import functools

import jax
import jax.numpy as jnp
from jax import lax
from jax.experimental import pallas as pl
from jax.experimental.pallas import tpu as pltpu
from jax.experimental.pallas import tpu_sc as plsc


def _make_sc_gather(n_rows: int, emb: int, num_cores: int, num_subcores: int):
    nw = num_cores * num_subcores
    rows_per_w = max(8, n_rows // nw)
    n_active = n_rows // rows_per_w
    mesh = plsc.VectorSubcoreMesh(core_axis_name="c", subcore_axis_name="s")

    @functools.partial(
        pl.kernel,
        mesh=mesh,
        out_type=jax.ShapeDtypeStruct((n_rows, emb), jnp.float32),
        scratch_types=[
            pltpu.VMEM((rows_per_w,), jnp.int32),
            pltpu.VMEM((rows_per_w, emb), jnp.float32),
            pltpu.SemaphoreType.DMA,
        ],
        compiler_params=pltpu.CompilerParams(use_tc_tiling_on_sc=False),
    )
    def gather(idx_hbm, table_hbm, pe_hbm, idx_v, rows_v, sem):
        wid = lax.axis_index("s") * num_cores + lax.axis_index("c")

        @pl.when(wid < n_active)
        def _():
            base = wid * rows_per_w
            pltpu.sync_copy(idx_hbm.at[pl.ds(base, rows_per_w)], idx_v)
            pltpu.async_copy(table_hbm.at[idx_v], rows_v, sem).wait()
            pltpu.sync_copy(rows_v, pe_hbm.at[pl.ds(base, rows_per_w)])

    return gather


def _make_add(rows: int, e: int, n: int, ch: int, nbuf: int):
    nchunks = rows // ch

    def body(x_hbm, pe_vmem, o_hbm, inb, outb, insems, outsems):
        def in_copy(c):
            slot = c % nbuf
            return pltpu.make_async_copy(
                x_hbm.at[pl.ds(c * ch, ch)], inb.at[slot], insems.at[slot])

        def out_copy(c):
            slot = c % nbuf
            return pltpu.make_async_copy(
                outb.at[slot], o_hbm.at[pl.ds(c * ch, ch)], outsems.at[slot])

        for c in range(nbuf):
            in_copy(c).start()
        for c in range(nchunks):
            slot = c % nbuf
            in_copy(c).wait()
            if c >= nbuf:
                out_copy(c - nbuf).wait()
            outb[slot] = inb[slot] + pe_vmem[...][None, :, :]
            out_copy(c).start(priority=1)
            if c + nbuf < nchunks:
                in_copy(c + nbuf).start()
        for c in range(nchunks - nbuf, nchunks):
            out_copy(c).wait()

    return body


def kernel(x, input_channels, pos_embed):
    B, N, P, E = x.shape
    input_channels = input_channels.astype(jnp.int32)

    info = plsc.get_sparse_core_info()
    gather = _make_sc_gather(N, E, info.num_cores, info.num_subcores)
    pe = gather(input_channels, pos_embed)

    xt = jnp.transpose(x, (0, 2, 3, 1)).reshape(B * P, E, N)
    pe_t = pe.T

    CH = 50
    NBUF = 4
    out_t = pl.pallas_call(
        _make_add(B * P, E, N, CH, NBUF),
        in_specs=[
            pl.BlockSpec(memory_space=pltpu.MemorySpace.HBM),
            pl.BlockSpec(memory_space=pltpu.MemorySpace.VMEM),
        ],
        out_specs=pl.BlockSpec(memory_space=pltpu.MemorySpace.HBM),
        out_shape=jax.ShapeDtypeStruct((B * P, E, N), jnp.float32),
        scratch_shapes=[
            pltpu.VMEM((NBUF, CH, E, N), jnp.float32),
            pltpu.VMEM((NBUF, CH, E, N), jnp.float32),
            pltpu.SemaphoreType.DMA((NBUF,)),
            pltpu.SemaphoreType.DMA((NBUF,)),
        ],
    )(xt, pe_t)
    return jnp.transpose(out_t.reshape(B, P, E, N), (0, 3, 1, 2))
